# Initial kernel scaffold; baseline (speedup 1.0000x reference)
import jax
import jax.numpy as jnp
from jax.experimental import pallas as pl


def kernel(latent_z, edge_idx, W1a, b1a, W1b, b1b, W2a, b2a, W2b, b2b, Wc1, bc1, Wc2, bc2):
    raise NotImplementedError("write your pallas kernel here")


# speedup vs baseline: 1.2356x; 1.2356x over previous
import functools

import jax
import jax.numpy as jnp
from jax import lax
from jax.experimental import pallas as pl
from jax.experimental.pallas import tpu as pltpu
from jax.experimental.pallas import tpu_sc as plsc

_N = 19
_E = 342
_LAT = 512
_HID = 2048
_T = 4096
_APAD = 384
_EPAD = 384
_IROWS = 4
_ICOLS = 96
_W = 128


def _build_m(edge_idx):
    mesh = plsc.VectorSubcoreMesh(core_axis_name="c", subcore_axis_name="s")

    @functools.partial(
        pl.kernel,
        mesh=mesh,
        out_type=jax.ShapeDtypeStruct((_APAD, _W), jnp.float32),
        scratch_types=[
            pltpu.VMEM((_EPAD,), jnp.int32),
            pltpu.VMEM((_EPAD,), jnp.int32),
            pltpu.VMEM((_ICOLS,), jnp.int32),
            pltpu.VMEM((_ICOLS,), jnp.int32),
            pltpu.VMEM((_ICOLS,), jnp.int32),
            pltpu.VMEM((_ICOLS,), jnp.int32),
            pltpu.VMEM((_ICOLS, _W), jnp.float32),
            pltpu.VMEM_SHARED((_APAD, _W), jnp.float32),
        ],
    )
    def sc_count(src_hbm, dst_hbm, zeros_hbm, ones_hbm, out_hbm,
                 src_v, dst_v, idx0, idx1, idx2, idx3, ones_v, a_sh):
        wid = lax.axis_index("s") * 2 + lax.axis_index("c")

        @pl.when(wid == 0)
        def _():
            pltpu.sync_copy(src_hbm, src_v.at[pl.ds(0, _E)])
            pltpu.sync_copy(dst_hbm, dst_v.at[pl.ds(0, _E)])
            pltpu.sync_copy(zeros_hbm, a_sh)
            pltpu.sync_copy(ones_hbm, ones_v)
            lane = lax.iota(jnp.int32, 16)
            for j, idx_v in enumerate((idx0, idx1, idx2, idx3)):
                for t in range(_ICOLS // 16):
                    base = j * _ICOLS + t * 16
                    s = src_v[pl.ds(base, 16)]
                    d = dst_v[pl.ds(base, 16)]
                    valid = (base + lane) < _E
                    idx_v[pl.ds(t * 16, 16)] = jnp.where(
                        valid, d * _N + s, _N * _N)
            for idx_v in (idx0, idx1, idx2, idx3):
                pltpu.sync_copy(ones_v, a_sh.at[idx_v], add=True)
            pltpu.sync_copy(a_sh, out_hbm)

    counts = sc_count(edge_idx[0], edge_idx[1],
                      jnp.zeros((_APAD, _W), jnp.float32),
                      jnp.ones((_ICOLS, _W), jnp.float32))
    a = counts[:_N * _N, 0].reshape(_N, _N)
    return a + jnp.eye(_N, dtype=jnp.float32)


def _mm_body(x_ref, w_ref, o_ref):
    o_ref[...] = jnp.dot(x_ref[...], w_ref[...],
                         preferred_element_type=jnp.float32)


def _stage1_body(m_ref, y1_ref, b1a_ref, w1b_ref, b1b_ref, o_ref):
    h1 = jnp.maximum(
        jnp.dot(m_ref[...], y1_ref[...], preferred_element_type=jnp.float32)
        + b1a_ref[...], 0.0)
    o_ref[...] = jnp.dot(h1, w1b_ref[...],
                         preferred_element_type=jnp.float32) + b1b_ref[...]


def _stage2_body(m_ref, g_ref, w2a_ref, b2a_ref, o_ref):
    x2 = jnp.dot(m_ref[...], jnp.maximum(g_ref[...], 0.0),
                 preferred_element_type=jnp.float32)
    o_ref[...] = jnp.maximum(
        jnp.dot(x2, w2a_ref[...], preferred_element_type=jnp.float32)
        + b2a_ref[...], 0.0)


def _stage3_body(h2_ref, w2b_ref, b2b_ref, o_ref):
    o_ref[...] = jnp.dot(h2_ref[...], w2b_ref[...],
                         preferred_element_type=jnp.float32) + b2b_ref[...]


def _cls_body(flat_ref, wc1_ref, bc1_ref, wc2_ref, bc2_ref, o_ref, acc_ref):
    k = pl.program_id(0)
    part = jnp.dot(flat_ref[...], wc1_ref[...],
                   preferred_element_type=jnp.float32)

    @pl.when(k == 0)
    def _():
        acc_ref[...] = part

    @pl.when(k > 0)
    def _():
        acc_ref[...] += part

    @pl.when(k == pl.num_programs(0) - 1)
    def _():
        z = acc_ref[...] + bc1_ref[...]
        p = jnp.dot(z, wc2_ref[...],
                    preferred_element_type=jnp.float32) + bc2_ref[...]
        o_ref[...] = jax.nn.sigmoid(p)


_ARB = pltpu.CompilerParams(dimension_semantics=("arbitrary",))


def kernel(latent_z, edge_idx, W1a, b1a, W1b, b1b, W2a, b2a, W2b, b2b,
           Wc1, bc1, Wc2, bc2):
    m = _build_m(edge_idx)

    y1 = pl.pallas_call(
        _mm_body,
        out_shape=jax.ShapeDtypeStruct((_N, _HID), jnp.float32),
    )(latent_z, W1a)

    nb1 = 4
    c1 = _HID // nb1
    g = pl.pallas_call(
        _stage1_body,
        grid=(nb1,),
        in_specs=[
            pl.BlockSpec((_N, _N), lambda j: (0, 0)),
            pl.BlockSpec((_N, _HID), lambda j: (0, 0)),
            pl.BlockSpec((1, _HID), lambda j: (0, 0)),
            pl.BlockSpec((_HID, c1), lambda j: (0, j)),
            pl.BlockSpec((1, c1), lambda j: (0, j)),
        ],
        out_specs=pl.BlockSpec((_N, c1), lambda j: (0, j)),
        out_shape=jax.ShapeDtypeStruct((_N, _HID), jnp.float32),
        compiler_params=_ARB,
    )(m, y1, b1a.reshape(1, -1), W1b, b1b.reshape(1, -1))

    nb2 = 8
    c2 = _T // nb2
    h2 = pl.pallas_call(
        _stage2_body,
        grid=(nb2,),
        in_specs=[
            pl.BlockSpec((_N, _N), lambda j: (0, 0)),
            pl.BlockSpec((_N, _HID), lambda j: (0, 0)),
            pl.BlockSpec((_HID, c2), lambda j: (0, j)),
            pl.BlockSpec((1, c2), lambda j: (0, j)),
        ],
        out_specs=pl.BlockSpec((_N, c2), lambda j: (0, j)),
        out_shape=jax.ShapeDtypeStruct((_N, _T), jnp.float32),
        compiler_params=_ARB,
    )(m, g, W2a, b2a.reshape(1, -1))

    nb3 = 8
    c3 = _T // nb3
    gin = pl.pallas_call(
        _stage3_body,
        grid=(nb3,),
        in_specs=[
            pl.BlockSpec((_N, _T), lambda j: (0, 0)),
            pl.BlockSpec((_T, c3), lambda j: (0, j)),
            pl.BlockSpec((1, c3), lambda j: (0, j)),
        ],
        out_specs=pl.BlockSpec((_N, c3), lambda j: (0, j)),
        out_shape=jax.ShapeDtypeStruct((_N, _T), jnp.float32),
        compiler_params=_ARB,
    )(h2, W2b, b2b.reshape(1, -1))

    flat = gin.reshape(1, _N * _T)
    pred = pl.pallas_call(
        _cls_body,
        grid=(_N,),
        in_specs=[
            pl.BlockSpec((1, _T), lambda k: (0, k)),
            pl.BlockSpec((_T, _LAT), lambda k: (k, 0)),
            pl.BlockSpec((1, _LAT), lambda k: (0, 0)),
            pl.BlockSpec((_LAT, 1), lambda k: (0, 0)),
            pl.BlockSpec((1, 1), lambda k: (0, 0)),
        ],
        out_specs=pl.BlockSpec((1, 1), lambda k: (0, 0)),
        out_shape=jax.ShapeDtypeStruct((1, 1), jnp.float32),
        scratch_shapes=[pltpu.VMEM((1, _LAT), jnp.float32)],
        compiler_params=_ARB,
    )(flat, Wc1, bc1.reshape(1, -1), Wc2, bc2.reshape(1, -1))

    return (pred.reshape(1), gin)
